# Initial kernel scaffold; baseline (speedup 1.0000x reference)
import functools

import jax
import jax.numpy as jnp
from jax.experimental import pallas as pl
from jax.experimental.pallas import tpu as pltpu


def _body(wl_ref, sl_ref, pos_ref, feats_ref, ptab_ref, out_ref):
    wl = wl_ref[0, :].astype(jnp.int32)
    seq_len = sl_ref[0, 0]
    pos = pos_ref[0, :].astype(jnp.int32)
    feats = feats_ref[0]
    ptab = ptab_ref[...]

    W = wl.shape[0]
    S = feats.shape[0]
    PV = ptab.shape[0]

    nxt = jnp.concatenate([wl[1:], jnp.zeros((1,), jnp.int32)])
    end = jnp.where(nxt == 0, seq_len, nxt)
    start = jnp.clip(wl, 0, S)
    end = jnp.clip(end, 0, S)
    lo = jnp.minimum(start, end)
    hi = jnp.maximum(start, end)

    jidx = jax.lax.broadcasted_iota(jnp.int32, (W, 1), 0)[:, 0]
    valid = ~((wl == 0) & (jidx != 0))
    coef = jnp.where(end > start,
                     1.0 / jnp.maximum(end - start, 1).astype(jnp.float32),
                     -1.0)
    coef = jnp.where(valid, coef, 0.0)

    sidx = jax.lax.broadcasted_iota(jnp.int32, (W, S), 1)
    m = ((sidx >= lo[:, None]) & (sidx < hi[:, None])).astype(jnp.float32)
    seg = jax.lax.dot(m, feats, preferred_element_type=jnp.float32)

    pvidx = jax.lax.broadcasted_iota(jnp.int32, (W, PV), 1)
    onehot = (pos[:, None] == pvidx).astype(jnp.float32)
    pemb = jax.lax.dot(onehot, ptab, preferred_element_type=jnp.float32)

    out_ref[0] = seg * coef[:, None] + pemb


def kernel(feats, word_lens, seq_len, pos, pos_table):
    B, S, D = feats.shape
    W = word_lens.shape[1]
    PV = pos_table.shape[0]
    sl2 = seq_len.reshape(B, 1).astype(jnp.int32)
    grid = (B,)
    return pl.pallas_call(
        _body,
        grid=grid,
        in_specs=[
            pl.BlockSpec((1, W), lambda i: (i, 0)),
            pl.BlockSpec((1, 1), lambda i: (i, 0)),
            pl.BlockSpec((1, W), lambda i: (i, 0)),
            pl.BlockSpec((1, S, D), lambda i: (i, 0, 0)),
            pl.BlockSpec((PV, D), lambda i: (0, 0)),
        ],
        out_specs=pl.BlockSpec((1, W, D), lambda i: (i, 0, 0)),
        out_shape=jax.ShapeDtypeStruct((B, W, D), jnp.float32),
    )(word_lens, sl2, pos, feats, pos_table)


# speedup vs baseline: 8.5243x; 8.5243x over previous
import functools

import jax
import jax.numpy as jnp
from jax.experimental import pallas as pl
from jax.experimental.pallas import tpu as pltpu


def _body(wl_ref, sl_ref, pos_ref, feats_ref, ptab_ref, out_ref):
    wl = wl_ref[0, 0, :].astype(jnp.int32)
    seq_len = sl_ref[0, 0, 0]
    pos = pos_ref[0, 0, :].astype(jnp.int32)
    feats = feats_ref[0]
    ptab = ptab_ref[...]

    W = wl.shape[0]
    S = feats.shape[0]
    PV = ptab.shape[0]

    nxt = jnp.concatenate([wl[1:], jnp.zeros((1,), jnp.int32)])
    end = jnp.where(nxt == 0, seq_len, nxt)
    start = jnp.clip(wl, 0, S)
    end = jnp.clip(end, 0, S)
    lo = jnp.minimum(start, end)
    hi = jnp.maximum(start, end)

    jidx = jax.lax.broadcasted_iota(jnp.int32, (W, 1), 0)[:, 0]
    valid = ~((wl == 0) & (jidx != 0))
    coef = jnp.where(end > start,
                     1.0 / jnp.maximum(end - start, 1).astype(jnp.float32),
                     -1.0)
    coef = jnp.where(valid, coef, 0.0)

    sidx = jax.lax.broadcasted_iota(jnp.int32, (W, S), 1)
    m = ((sidx >= lo[:, None]) & (sidx < hi[:, None])).astype(jnp.float32)
    seg = jax.lax.dot(m, feats, preferred_element_type=jnp.float32)

    pvidx = jax.lax.broadcasted_iota(jnp.int32, (W, PV), 1)
    onehot = (pos[:, None] == pvidx).astype(jnp.float32)
    pemb = jax.lax.dot(onehot, ptab, preferred_element_type=jnp.float32)

    out_ref[0] = seg * coef[:, None] + pemb


def kernel(feats, word_lens, seq_len, pos, pos_table):
    B, S, D = feats.shape
    W = word_lens.shape[1]
    PV = pos_table.shape[0]
    wl3 = word_lens.reshape(B, 1, W).astype(jnp.int32)
    pos3 = pos.reshape(B, 1, W).astype(jnp.int32)
    sl3 = seq_len.reshape(B, 1, 1).astype(jnp.int32)
    grid = (B,)
    return pl.pallas_call(
        _body,
        grid=grid,
        in_specs=[
            pl.BlockSpec((1, 1, W), lambda i: (i, 0, 0)),
            pl.BlockSpec((1, 1, 1), lambda i: (i, 0, 0)),
            pl.BlockSpec((1, 1, W), lambda i: (i, 0, 0)),
            pl.BlockSpec((1, S, D), lambda i: (i, 0, 0)),
            pl.BlockSpec((PV, D), lambda i: (0, 0)),
        ],
        out_specs=pl.BlockSpec((1, W, D), lambda i: (i, 0, 0)),
        out_shape=jax.ShapeDtypeStruct((B, W, D), jnp.float32),
    )(wl3, sl3, pos3, feats, pos_table)
